# Initial kernel scaffold; baseline (speedup 1.0000x reference)
import jax
import jax.numpy as jnp
from jax.experimental import pallas as pl


def kernel(input_rep, final_rep, graph_index, W_lin, b_lin, W_last, b_last):
    raise NotImplementedError("write your pallas kernel here")


# speedup vs baseline: 1.9866x; 1.9866x over previous
import functools

import jax
import jax.numpy as jnp
from jax import lax
from jax.experimental import pallas as pl
from jax.experimental.pallas import tpu as pltpu
from jax.experimental.pallas import tpu_sc as plsc

N_NODES = 100000
NUM_GRAPHS = 4096
CH = 128

NW = 16
N_PAD = 102400
ROWS_W = N_PAD // NW
CHUNK = 128
NCHUNK = ROWS_W // CHUNK
SEG_W = NUM_GRAPHS // NW

BLK = 800
N_BLOCKS = N_PAD // BLK
REAL_BLOCKS = N_NODES // BLK


def _tc_body(x1_ref, x2_ref, w1_ref, w2_ref, bl_ref, wl_ref, bb_ref, out_ref):
    i = pl.program_id(0)

    @pl.when(i < REAL_BLOCKS)
    def _compute():
        x1 = x1_ref[...]
        x2 = x2_ref[...]
        z = (jnp.dot(x1, w1_ref[...], preferred_element_type=jnp.float32)
             + jnp.dot(x2, w2_ref[...], preferred_element_type=jnp.float32)
             + bl_ref[...])
        h = (jnp.dot(x2, wl_ref[...], preferred_element_type=jnp.float32)
             + bb_ref[...])
        out_ref[...] = jax.nn.sigmoid(z) * h

    @pl.when(i >= REAL_BLOCKS)
    def _pad():
        out_ref[...] = jnp.zeros_like(out_ref)


def _tc_gated(input_rep, final_rep, w1, w2, b_lin, w_last, b_last):
    clamp = lambda i: (jnp.minimum(i, REAL_BLOCKS - 1), 0)
    full = lambda i: (0, 0)
    return pl.pallas_call(
        _tc_body,
        grid=(N_BLOCKS,),
        in_specs=[
            pl.BlockSpec((BLK, CH), clamp),
            pl.BlockSpec((BLK, CH), clamp),
            pl.BlockSpec((CH, CH), full),
            pl.BlockSpec((CH, CH), full),
            pl.BlockSpec((1, CH), full),
            pl.BlockSpec((CH, CH), full),
            pl.BlockSpec((1, CH), full),
        ],
        out_specs=pl.BlockSpec((BLK, CH), lambda i: (i, 0)),
        out_shape=jax.ShapeDtypeStruct((N_PAD, CH), jnp.float32),
        compiler_params=pltpu.CompilerParams(
            dimension_semantics=("arbitrary",),
        ),
    )(input_rep, final_rep, w1, w2, b_lin, w_last, b_last)


@functools.cache
def _make_sc_segment_sum():
    mesh = plsc.VectorSubcoreMesh(
        core_axis_name="c", subcore_axis_name="s",
        num_cores=1, num_subcores=NW)
    return pl.kernel(
        _sc_body,
        out_type=jax.ShapeDtypeStruct((NUM_GRAPHS, CH), jnp.float32),
        mesh=mesh,
        scratch_types=[
            pltpu.VMEM((NCHUNK, CHUNK), jnp.int32),
            pltpu.VMEM((CHUNK, CH), jnp.float32),
            pltpu.VMEM((SEG_W, CH), jnp.float32),
            pltpu.VMEM_SHARED((NUM_GRAPHS, CH), jnp.float32),
            pltpu.SemaphoreType.DMA,
        ],
    )


def _sc_body(g_hbm, gi_hbm, zero_hbm, out_hbm,
             idx_v, rows_v, seg_v, acc_sh, sem):
    w = lax.axis_index("s")
    seg_base = w * SEG_W

    pltpu.sync_copy(zero_hbm.at[pl.ds(seg_base, SEG_W)], seg_v)
    pltpu.sync_copy(seg_v, acc_sh.at[pl.ds(seg_base, SEG_W)])

    pltpu.sync_copy(gi_hbm.at[w], idx_v)
    plsc.subcore_barrier()

    def body(j, carry):
        pltpu.sync_copy(
            g_hbm.at[pl.ds(w * ROWS_W + j * CHUNK, CHUNK)], rows_v)
        pltpu.sync_copy(rows_v, acc_sh.at[idx_v.at[j]], add=True)
        return carry

    lax.fori_loop(0, NCHUNK, body, 0)
    plsc.subcore_barrier()

    pltpu.sync_copy(acc_sh.at[pl.ds(seg_base, SEG_W)], seg_v)
    pltpu.sync_copy(seg_v, out_hbm.at[pl.ds(seg_base, SEG_W)])


def kernel(input_rep, final_rep, graph_index, W_lin, b_lin, W_last, b_last):
    gi = graph_index.astype(jnp.int32)
    gi = jnp.pad(gi, (0, N_PAD - N_NODES)).reshape(NW, NCHUNK, CHUNK)
    w1 = W_lin[:CH]
    w2 = W_lin[CH:]
    g = _tc_gated(input_rep, final_rep, w1, w2,
                  b_lin.reshape(1, CH), W_last, b_last.reshape(1, CH))
    zeros = jnp.zeros((NUM_GRAPHS, CH), jnp.float32)
    return _make_sc_segment_sum()(g, gi, zeros)


# speedup vs baseline: 2.3768x; 1.1964x over previous
import functools

import jax
import jax.numpy as jnp
from jax import lax
from jax.experimental import pallas as pl
from jax.experimental.pallas import tpu as pltpu
from jax.experimental.pallas import tpu_sc as plsc

N_NODES = 100000
NUM_GRAPHS = 4096
CH = 128

NC = 2
NS = 16
NW = NC * NS
N_PAD = 102400
ROWS_W = N_PAD // NW
CHUNK = 64
NCHUNK = ROWS_W // CHUNK
SEG_S = NUM_GRAPHS // NS

BLK = 800
N_BLOCKS = N_PAD // BLK
REAL_BLOCKS = N_NODES // BLK


def _tc_body(x1_ref, x2_ref, w1_ref, w2_ref, bl_ref, wl_ref, bb_ref, out_ref):
    i = pl.program_id(0)

    @pl.when(i < REAL_BLOCKS)
    def _compute():
        x1 = x1_ref[...]
        x2 = x2_ref[...]
        z = (jnp.dot(x1, w1_ref[...], preferred_element_type=jnp.float32)
             + jnp.dot(x2, w2_ref[...], preferred_element_type=jnp.float32)
             + bl_ref[...])
        h = (jnp.dot(x2, wl_ref[...], preferred_element_type=jnp.float32)
             + bb_ref[...])
        out_ref[...] = jax.nn.sigmoid(z) * h

    @pl.when(i >= REAL_BLOCKS)
    def _pad():
        out_ref[...] = jnp.zeros_like(out_ref)


def _tc_gated(input_rep, final_rep, w1, w2, b_lin, w_last, b_last):
    clamp = lambda i: (jnp.minimum(i, REAL_BLOCKS - 1), 0)
    full = lambda i: (0, 0)
    return pl.pallas_call(
        _tc_body,
        grid=(N_BLOCKS,),
        in_specs=[
            pl.BlockSpec((BLK, CH), clamp),
            pl.BlockSpec((BLK, CH), clamp),
            pl.BlockSpec((CH, CH), full),
            pl.BlockSpec((CH, CH), full),
            pl.BlockSpec((1, CH), full),
            pl.BlockSpec((CH, CH), full),
            pl.BlockSpec((1, CH), full),
        ],
        out_specs=pl.BlockSpec((BLK, CH), lambda i: (i, 0)),
        out_shape=jax.ShapeDtypeStruct((N_PAD, CH), jnp.float32),
        compiler_params=pltpu.CompilerParams(
            dimension_semantics=("arbitrary",),
        ),
    )(input_rep, final_rep, w1, w2, b_lin, w_last, b_last)


def _sc_body(g_hbm, gi_hbm, zero_hbm, out_hbm,
             idx_v, rows_v, seg_v, acc_sh, semA, semB):
    c = lax.axis_index("c")
    s = lax.axis_index("s")
    w = c * NS + s
    seg_base = s * SEG_S

    pltpu.sync_copy(zero_hbm.at[pl.ds(seg_base, SEG_S)], seg_v)
    pltpu.sync_copy(seg_v, acc_sh.at[pl.ds(seg_base, SEG_S)])

    pltpu.sync_copy(gi_hbm.at[w], idx_v)
    plsc.subcore_barrier()

    row0 = w * ROWS_W

    def start_g(j, b, sem):
        pltpu.async_copy(
            g_hbm.at[pl.ds(row0 + j * CHUNK, CHUNK)], rows_v.at[b], sem)

    def wait_g(j, b, sem):
        pltpu.make_async_copy(
            g_hbm.at[pl.ds(row0 + j * CHUNK, CHUNK)], rows_v.at[b],
            sem).wait()

    start_g(0, 0, semA)

    def pair(i, carry):
        j0 = 2 * i
        j1 = j0 + 1
        wait_g(j0, 0, semA)
        start_g(j1, 1, semB)
        pltpu.sync_copy(rows_v.at[0], acc_sh.at[idx_v.at[j0]], add=True)
        wait_g(j1, 1, semB)

        @pl.when(j1 + 1 < NCHUNK)
        def _():
            start_g(j1 + 1, 0, semA)

        pltpu.sync_copy(rows_v.at[1], acc_sh.at[idx_v.at[j1]], add=True)
        return carry

    lax.fori_loop(0, NCHUNK // 2, pair, 0)
    plsc.subcore_barrier()

    pltpu.sync_copy(acc_sh.at[pl.ds(seg_base, SEG_S)], seg_v)
    pltpu.sync_copy(seg_v, out_hbm.at[c, pl.ds(seg_base, SEG_S)])


@functools.cache
def _make_sc_segment_sum():
    mesh = plsc.VectorSubcoreMesh(
        core_axis_name="c", subcore_axis_name="s",
        num_cores=NC, num_subcores=NS)
    return pl.kernel(
        _sc_body,
        out_type=jax.ShapeDtypeStruct((NC, NUM_GRAPHS, CH), jnp.float32),
        mesh=mesh,
        scratch_types=[
            pltpu.VMEM((NCHUNK, CHUNK), jnp.int32),
            pltpu.VMEM((2, CHUNK, CH), jnp.float32),
            pltpu.VMEM((SEG_S, CH), jnp.float32),
            pltpu.VMEM_SHARED((NUM_GRAPHS, CH), jnp.float32),
            pltpu.SemaphoreType.DMA,
            pltpu.SemaphoreType.DMA,
        ],
    )


def _merge_body(p_ref, out_ref):
    out_ref[...] = p_ref[0] + p_ref[1]


def _tc_merge(partials):
    blk = 512
    return pl.pallas_call(
        _merge_body,
        grid=(NUM_GRAPHS // blk,),
        in_specs=[pl.BlockSpec((NC, blk, CH), lambda i: (0, i, 0))],
        out_specs=pl.BlockSpec((blk, CH), lambda i: (i, 0)),
        out_shape=jax.ShapeDtypeStruct((NUM_GRAPHS, CH), jnp.float32),
    )(partials)


def kernel(input_rep, final_rep, graph_index, W_lin, b_lin, W_last, b_last):
    gi = graph_index.astype(jnp.int32)
    gi = jnp.pad(gi, (0, N_PAD - N_NODES)).reshape(NW, NCHUNK, CHUNK)
    w1 = W_lin[:CH]
    w2 = W_lin[CH:]
    g = _tc_gated(input_rep, final_rep, w1, w2,
                  b_lin.reshape(1, CH), W_last, b_last.reshape(1, CH))
    zeros = jnp.zeros((NUM_GRAPHS, CH), jnp.float32)
    partials = _make_sc_segment_sum()(g, gi, zeros)
    return _tc_merge(partials)


# speedup vs baseline: 2.7194x; 1.1441x over previous
import functools

import jax
import jax.numpy as jnp
from jax import lax
from jax.experimental import pallas as pl
from jax.experimental.pallas import tpu as pltpu
from jax.experimental.pallas import tpu_sc as plsc

N_NODES = 100000
NUM_GRAPHS = 4096
CH = 128

NC = 2
NS = 16
NW = NC * NS
N_PAD = 102400
SLABS = 4
SLAB_ROWS = N_PAD // SLABS
ROWS_W = SLAB_ROWS // NW
CHUNK = 80
NCHUNK = ROWS_W // CHUNK
SEG_S = NUM_GRAPHS // NS

BLK = 800
BLOCKS_SLAB = SLAB_ROWS // BLK
REAL_BLOCKS = N_NODES // BLK


def _tc_body_for_slab(slab):
    def body(x1_ref, x2_ref, w1_ref, w2_ref, bl_ref, wl_ref, bb_ref, out_ref):
        gi = slab * BLOCKS_SLAB + pl.program_id(0)

        @pl.when(gi < REAL_BLOCKS)
        def _compute():
            x1 = x1_ref[...]
            x2 = x2_ref[...]
            z = (jnp.dot(x1, w1_ref[...], preferred_element_type=jnp.float32)
                 + jnp.dot(x2, w2_ref[...], preferred_element_type=jnp.float32)
                 + bl_ref[...])
            h = (jnp.dot(x2, wl_ref[...], preferred_element_type=jnp.float32)
                 + bb_ref[...])
            out_ref[...] = jax.nn.sigmoid(z) * h

        @pl.when(gi >= REAL_BLOCKS)
        def _pad():
            out_ref[...] = jnp.zeros_like(out_ref)

    return body


def _tc_gated_slab(slab, input_rep, final_rep, w1, w2, b_lin, w_last, b_last):
    clamp = lambda i: (jnp.minimum(slab * BLOCKS_SLAB + i, REAL_BLOCKS - 1), 0)
    full = lambda i: (0, 0)
    return pl.pallas_call(
        _tc_body_for_slab(slab),
        grid=(BLOCKS_SLAB,),
        in_specs=[
            pl.BlockSpec((BLK, CH), clamp),
            pl.BlockSpec((BLK, CH), clamp),
            pl.BlockSpec((CH, CH), full),
            pl.BlockSpec((CH, CH), full),
            pl.BlockSpec((1, CH), full),
            pl.BlockSpec((CH, CH), full),
            pl.BlockSpec((1, CH), full),
        ],
        out_specs=pl.BlockSpec((BLK, CH), lambda i: (i, 0)),
        out_shape=jax.ShapeDtypeStruct((SLAB_ROWS, CH), jnp.float32),
        compiler_params=pltpu.CompilerParams(
            dimension_semantics=("arbitrary",),
        ),
        name=f"tc_gated_slab{slab}",
    )(input_rep, final_rep, w1, w2, b_lin, w_last, b_last)


def _sc_body(g_hbm, gi_hbm, init_hbm, out_hbm,
             idx_v, rows_v, seg_v, acc_sh, semA, semB):
    c = lax.axis_index("c")
    s = lax.axis_index("s")
    w = c * NS + s
    seg_base = s * SEG_S

    pltpu.sync_copy(init_hbm.at[c, pl.ds(seg_base, SEG_S)], seg_v)
    pltpu.sync_copy(seg_v, acc_sh.at[pl.ds(seg_base, SEG_S)])

    pltpu.sync_copy(gi_hbm.at[w], idx_v)
    plsc.subcore_barrier()

    row0 = w * ROWS_W

    def start_g(j, b, sem):
        pltpu.async_copy(
            g_hbm.at[pl.ds(row0 + j * CHUNK, CHUNK)], rows_v.at[b], sem)

    def wait_g(j, b, sem):
        pltpu.make_async_copy(
            g_hbm.at[pl.ds(row0 + j * CHUNK, CHUNK)], rows_v.at[b],
            sem).wait()

    start_g(0, 0, semA)

    def pair(i, carry):
        j0 = 2 * i
        j1 = j0 + 1
        wait_g(j0, 0, semA)
        start_g(j1, 1, semB)
        pltpu.sync_copy(rows_v.at[0], acc_sh.at[idx_v.at[j0]], add=True)
        wait_g(j1, 1, semB)

        @pl.when(j1 + 1 < NCHUNK)
        def _():
            start_g(j1 + 1, 0, semA)

        pltpu.sync_copy(rows_v.at[1], acc_sh.at[idx_v.at[j1]], add=True)
        return carry

    lax.fori_loop(0, NCHUNK // 2, pair, 0)
    plsc.subcore_barrier()

    pltpu.sync_copy(acc_sh.at[pl.ds(seg_base, SEG_S)], seg_v)
    pltpu.sync_copy(seg_v, out_hbm.at[c, pl.ds(seg_base, SEG_S)])


@functools.cache
def _make_sc_segment_sum():
    mesh = plsc.VectorSubcoreMesh(
        core_axis_name="c", subcore_axis_name="s",
        num_cores=NC, num_subcores=NS)
    return pl.kernel(
        _sc_body,
        out_type=jax.ShapeDtypeStruct((NC, NUM_GRAPHS, CH), jnp.float32),
        mesh=mesh,
        scratch_types=[
            pltpu.VMEM((NCHUNK, CHUNK), jnp.int32),
            pltpu.VMEM((2, CHUNK, CH), jnp.float32),
            pltpu.VMEM((SEG_S, CH), jnp.float32),
            pltpu.VMEM_SHARED((NUM_GRAPHS, CH), jnp.float32),
            pltpu.SemaphoreType.DMA,
            pltpu.SemaphoreType.DMA,
        ],
    )


def _merge_body(p_ref, out_ref):
    out_ref[...] = p_ref[0] + p_ref[1]


def _tc_merge(partials):
    blk = 512
    return pl.pallas_call(
        _merge_body,
        grid=(NUM_GRAPHS // blk,),
        in_specs=[pl.BlockSpec((NC, blk, CH), lambda i: (0, i, 0))],
        out_specs=pl.BlockSpec((blk, CH), lambda i: (i, 0)),
        out_shape=jax.ShapeDtypeStruct((NUM_GRAPHS, CH), jnp.float32),
    )(partials)


def kernel(input_rep, final_rep, graph_index, W_lin, b_lin, W_last, b_last):
    gi = graph_index.astype(jnp.int32)
    gi = jnp.pad(gi, (0, N_PAD - N_NODES)).reshape(SLABS, NW, NCHUNK, CHUNK)
    w1 = W_lin[:CH]
    w2 = W_lin[CH:]
    bl = b_lin.reshape(1, CH)
    bb = b_last.reshape(1, CH)

    sc_segsum = _make_sc_segment_sum()
    partials = jnp.zeros((NC, NUM_GRAPHS, CH), jnp.float32)
    for k in range(SLABS):
        g_k = _tc_gated_slab(k, input_rep, final_rep, w1, w2, bl, W_last, bb)
        partials = sc_segsum(g_k, gi[k], partials)
    return _tc_merge(partials)


# speedup vs baseline: 2.8691x; 1.0551x over previous
import functools

import jax
import jax.numpy as jnp
import numpy as np
from jax import lax
from jax.experimental import pallas as pl
from jax.experimental.pallas import tpu as pltpu
from jax.experimental.pallas import tpu_sc as plsc

N_NODES = 100000
NUM_GRAPHS = 4096
CH = 128

NC = 2
NS = 16
NW = NC * NS
N_PAD = 102400
SLABS = 4
SLAB_ROWS = N_PAD // SLABS
ROWS_W = SLAB_ROWS // NW
PROWS_W = ROWS_W // 2
CHUNK = 80
NCHUNK = ROWS_W // CHUNK
NITER = NCHUNK // 2
SEG_S = NUM_GRAPHS // NS

BLK = 800
PBLK = BLK // 2
BLOCKS_SLAB = SLAB_ROWS // BLK
REAL_BLOCKS = N_NODES // BLK

_HI_MASK = np.uint32(0xFFFF0000)
_ROUND = np.uint32(0x8000)
_SH16 = np.uint32(16)
_IHI_MASK = np.int32(-65536)
_ISH16 = np.int32(16)


def _tc_body_for_slab(slab):
    def body(x1_ref, x2_ref, w1_ref, w2_ref, bl_ref, wl_ref, bb_ref, out_ref):
        gi = slab * BLOCKS_SLAB + pl.program_id(0)

        @pl.when(gi < REAL_BLOCKS)
        def _compute():
            x1 = x1_ref[...]
            x2 = x2_ref[...]
            z = (jnp.dot(x1, w1_ref[...], preferred_element_type=jnp.float32)
                 + jnp.dot(x2, w2_ref[...], preferred_element_type=jnp.float32)
                 + bl_ref[...])
            h = (jnp.dot(x2, wl_ref[...], preferred_element_type=jnp.float32)
                 + bb_ref[...])
            g = jax.nn.sigmoid(z) * h
            ulo = lax.bitcast_convert_type(g[:PBLK], jnp.uint32)
            uhi = lax.bitcast_convert_type(g[PBLK:], jnp.uint32)
            lo16 = lax.shift_right_logical(ulo + _ROUND, _SH16)
            hi16 = (uhi + _ROUND) & _HI_MASK
            out_ref[...] = lax.bitcast_convert_type(hi16 | lo16, jnp.float32)

        @pl.when(gi >= REAL_BLOCKS)
        def _pad():
            out_ref[...] = jnp.zeros_like(out_ref)

    return body


def _tc_gated_slab(slab, input_rep, final_rep, W_lin, b_lin, w_last, b_last):
    clamp = lambda i: (jnp.minimum(slab * BLOCKS_SLAB + i, REAL_BLOCKS - 1), 0)
    full = lambda i: (0, 0)
    return pl.pallas_call(
        _tc_body_for_slab(slab),
        grid=(BLOCKS_SLAB,),
        in_specs=[
            pl.BlockSpec((BLK, CH), clamp),
            pl.BlockSpec((BLK, CH), clamp),
            pl.BlockSpec((CH, CH), full),
            pl.BlockSpec((CH, CH), lambda i: (1, 0)),
            pl.BlockSpec((1, CH), full),
            pl.BlockSpec((CH, CH), full),
            pl.BlockSpec((1, CH), full),
        ],
        out_specs=pl.BlockSpec((PBLK, CH), lambda i: (i, 0)),
        out_shape=jax.ShapeDtypeStruct((SLAB_ROWS // 2, CH), jnp.float32),
        compiler_params=pltpu.CompilerParams(
            dimension_semantics=("arbitrary",),
        ),
        name=f"tc_gated_slab{slab}",
    )(input_rep, final_rep, W_lin, W_lin, b_lin, w_last, b_last)


def _sc_body(g_hbm, gi_hbm, init_hbm, out_hbm,
             idx_v, pk_v, st_v, seg_v, acc_sh,
             semA, semB, semL0, semL1, semH0, semH1):
    c = lax.axis_index("c")
    s = lax.axis_index("s")
    w = c * NS + s
    seg_base = s * SEG_S

    pltpu.sync_copy(init_hbm.at[c, pl.ds(seg_base, SEG_S)], seg_v)
    pltpu.sync_copy(seg_v, acc_sh.at[pl.ds(seg_base, SEG_S)])

    pltpu.sync_copy(gi_hbm.at[w], idx_v)
    plsc.subcore_barrier()

    p0 = w * PROWS_W
    gsems = (semA, semB)
    lsems = (semL0, semL1)
    hsems = (semH0, semH1)

    def start_g(i, b):
        pltpu.async_copy(
            g_hbm.at[pl.ds(p0 + i * CHUNK, CHUNK)], pk_v.at[b], gsems[b])

    def wait_g(i, b):
        pltpu.make_async_copy(
            g_hbm.at[pl.ds(p0 + i * CHUNK, CHUNK)], pk_v.at[b],
            gsems[b]).wait()

    def scat(buf, j, sem):
        return pltpu.async_copy(buf, acc_sh.at[idx_v.at[j]], sem)

    def wait_scat(buf, j, sem):
        pltpu.make_async_copy(buf, acc_sh.at[idx_v.at[j]], sem).wait()

    start_g(0, 0)
    for i in range(NITER):
        b = i % 2
        wait_g(i, b)
        if i + 1 < NITER:
            start_g(i + 1, 1 - b)
        if i >= 2:
            wait_scat(st_v.at[b, 0], i - 2, lsems[b])
            wait_scat(st_v.at[b, 1], (i - 2) + NITER, hsems[b])

        def unpack_row(p, carry):
            for c4 in range(CH // 16):
                u = lax.bitcast_convert_type(
                    pk_v[b, p, pl.ds(c4 * 16, 16)], jnp.int32)
                lo = lax.bitcast_convert_type(
                    lax.shift_left(u, _ISH16), jnp.float32)
                hi = lax.bitcast_convert_type(u & _IHI_MASK, jnp.float32)
                st_v[b, 0, p, pl.ds(c4 * 16, 16)] = lo
                st_v[b, 1, p, pl.ds(c4 * 16, 16)] = hi
            return carry

        lax.fori_loop(0, CHUNK, unpack_row, 0)
        pltpu.async_copy(st_v.at[b, 0], acc_sh.at[idx_v.at[i]],
                         lsems[b], add=True)
        pltpu.async_copy(st_v.at[b, 1], acc_sh.at[idx_v.at[i + NITER]],
                         hsems[b], add=True)

    for i in (NITER - 2, NITER - 1):
        b = i % 2
        wait_scat(st_v.at[b, 0], i, lsems[b])
        wait_scat(st_v.at[b, 1], i + NITER, hsems[b])

    plsc.subcore_barrier()

    pltpu.sync_copy(acc_sh.at[pl.ds(seg_base, SEG_S)], seg_v)
    pltpu.sync_copy(seg_v, out_hbm.at[c, pl.ds(seg_base, SEG_S)])


@functools.cache
def _make_sc_segment_sum():
    mesh = plsc.VectorSubcoreMesh(
        core_axis_name="c", subcore_axis_name="s",
        num_cores=NC, num_subcores=NS)
    return pl.kernel(
        _sc_body,
        out_type=jax.ShapeDtypeStruct((NC, NUM_GRAPHS, CH), jnp.float32),
        mesh=mesh,
        scratch_types=[
            pltpu.VMEM((NCHUNK, CHUNK), jnp.int32),
            pltpu.VMEM((2, CHUNK, CH), jnp.float32),
            pltpu.VMEM((2, 2, CHUNK, CH), jnp.float32),
            pltpu.VMEM((SEG_S, CH), jnp.float32),
            pltpu.VMEM_SHARED((NUM_GRAPHS, CH), jnp.float32),
            pltpu.SemaphoreType.DMA,
            pltpu.SemaphoreType.DMA,
            pltpu.SemaphoreType.DMA,
            pltpu.SemaphoreType.DMA,
            pltpu.SemaphoreType.DMA,
            pltpu.SemaphoreType.DMA,
        ],
    )


def _merge_body(p_ref, out_ref):
    out_ref[...] = p_ref[0] + p_ref[1]


def _tc_merge(partials):
    blk = 512
    return pl.pallas_call(
        _merge_body,
        grid=(NUM_GRAPHS // blk,),
        in_specs=[pl.BlockSpec((NC, blk, CH), lambda i: (0, i, 0))],
        out_specs=pl.BlockSpec((blk, CH), lambda i: (i, 0)),
        out_shape=jax.ShapeDtypeStruct((NUM_GRAPHS, CH), jnp.float32),
    )(partials)


def kernel(input_rep, final_rep, graph_index, W_lin, b_lin, W_last, b_last):
    gi = graph_index.astype(jnp.int32)
    gi = jnp.pad(gi, (0, N_PAD - N_NODES)).reshape(SLABS, NW, NCHUNK, CHUNK)
    bl = b_lin.reshape(1, CH)
    bb = b_last.reshape(1, CH)

    sc_segsum = _make_sc_segment_sum()
    partials = jnp.zeros((NC, NUM_GRAPHS, CH), jnp.float32)
    for k in range(SLABS):
        g_k = _tc_gated_slab(k, input_rep, final_rep, W_lin, bl, W_last, bb)
        partials = sc_segsum(g_k, gi[k], partials)
    return _tc_merge(partials)


# speedup vs baseline: 3.2308x; 1.1261x over previous
import functools

import jax
import jax.numpy as jnp
import numpy as np
from jax import lax
from jax.experimental import pallas as pl
from jax.experimental.pallas import tpu as pltpu
from jax.experimental.pallas import tpu_sc as plsc

N_NODES = 100000
NUM_GRAPHS = 4096
CH = 128

NC = 2
NS = 16
NW = NC * NS
N_PAD = 102400
SLABS = 4
SLAB_ROWS = N_PAD // SLABS
ROWS_W = SLAB_ROWS // NW
CHUNK = 80
NITER = ROWS_W // CHUNK
SEG_S = NUM_GRAPHS // NS

BLK = 1600
PBLK = BLK // 2
BLOCKS_SLAB = SLAB_ROWS // BLK
LAST_BLK = N_NODES // BLK

_HI_MASK = np.uint32(0xFFFF0000)
_ROUND = np.uint32(0x8000)
_SH16 = np.uint32(16)
_IHI_MASK = np.int32(-65536)


def _tc_body_for_slab(slab):
    def body(x1_ref, x2_ref, w1_ref, w2_ref, bl_ref, wl_ref, bb_ref, out_ref):
        gb = slab * BLOCKS_SLAB + pl.program_id(0)
        x1 = x1_ref[...].astype(jnp.bfloat16)
        x2 = x2_ref[...].astype(jnp.bfloat16)
        z = (jnp.dot(x1, w1_ref[...].astype(jnp.bfloat16),
                     preferred_element_type=jnp.float32)
             + jnp.dot(x2, w2_ref[...].astype(jnp.bfloat16),
                       preferred_element_type=jnp.float32)
             + bl_ref[...])
        h = (jnp.dot(x2, wl_ref[...].astype(jnp.bfloat16),
                     preferred_element_type=jnp.float32)
             + bb_ref[...])
        g = jax.nn.sigmoid(z) * h
        rows = gb * BLK + lax.broadcasted_iota(jnp.int32, (BLK, 1), 0)
        g = jnp.where(rows < N_NODES, g, 0.0)
        ulo = lax.bitcast_convert_type(g[:PBLK], jnp.uint32)
        uhi = lax.bitcast_convert_type(g[PBLK:], jnp.uint32)
        lo16 = lax.shift_right_logical(ulo + _ROUND, _SH16)
        hi16 = (uhi + _ROUND) & _HI_MASK
        out_ref[...] = lax.bitcast_convert_type(hi16 | lo16, jnp.float32)

    return body


def _tc_gated_slab(slab, input_rep, final_rep, W_lin, b_lin, w_last, b_last):
    clamp = lambda i: (jnp.minimum(slab * BLOCKS_SLAB + i, LAST_BLK), 0)
    full = lambda i: (0, 0)
    return pl.pallas_call(
        _tc_body_for_slab(slab),
        grid=(BLOCKS_SLAB,),
        in_specs=[
            pl.BlockSpec((BLK, CH), clamp),
            pl.BlockSpec((BLK, CH), clamp),
            pl.BlockSpec((CH, CH), full),
            pl.BlockSpec((CH, CH), lambda i: (1, 0)),
            pl.BlockSpec((1, CH), full),
            pl.BlockSpec((CH, CH), full),
            pl.BlockSpec((1, CH), full),
        ],
        out_specs=pl.BlockSpec((PBLK, CH), lambda i: (i, 0)),
        out_shape=jax.ShapeDtypeStruct((SLAB_ROWS // 2, CH), jnp.float32),
        compiler_params=pltpu.CompilerParams(
            dimension_semantics=("arbitrary",),
        ),
        name=f"tc_gated_slab{slab}",
    )(input_rep, final_rep, W_lin, W_lin, b_lin, w_last, b_last)


def _sc_body(g_hbm, gi_hbm, init_hbm, out_hbm,
             idx_v, pk_v, st_v, seg_v, acc_sh,
             semA, semB, semL0, semL1):
    c = lax.axis_index("c")
    s = lax.axis_index("s")
    w = c * NS + s
    seg_base = s * SEG_S

    pltpu.sync_copy(init_hbm.at[c, pl.ds(seg_base, SEG_S)], seg_v)
    pltpu.sync_copy(seg_v, acc_sh.at[pl.ds(seg_base, SEG_S)])

    pltpu.sync_copy(gi_hbm.at[w], idx_v)
    plsc.subcore_barrier()

    pair = w // 2
    half = w - 2 * pair
    pbase = pair * PBLK
    shv = jnp.full((16,), (1 - half) * 16, jnp.int32)
    mask = jnp.full((16,), _IHI_MASK, jnp.int32)
    gsems = (semA, semB)
    ssems = (semL0, semL1)

    def start_g(j, b):
        pltpu.async_copy(
            g_hbm.at[pl.ds(pbase + j * CHUNK, CHUNK)], pk_v.at[b], gsems[b])

    def wait_g(j, b):
        pltpu.make_async_copy(
            g_hbm.at[pl.ds(pbase + j * CHUNK, CHUNK)], pk_v.at[b],
            gsems[b]).wait()

    def wait_scat(b, j):
        pltpu.make_async_copy(
            st_v.at[b], acc_sh.at[idx_v.at[j]], ssems[b]).wait()

    start_g(0, 0)
    for j in range(NITER):
        b = j % 2
        wait_g(j, b)
        if j + 1 < NITER:
            start_g(j + 1, 1 - b)
        if j >= 2:
            wait_scat(b, j - 2)

        def unpack_row(p, carry):
            for c4 in range(CH // 16):
                u = lax.bitcast_convert_type(
                    pk_v[b, p, pl.ds(c4 * 16, 16)], jnp.int32)
                v = lax.bitcast_convert_type(
                    lax.shift_left(u, shv) & mask, jnp.float32)
                st_v[b, p, pl.ds(c4 * 16, 16)] = v
            return carry

        lax.fori_loop(0, CHUNK, unpack_row, 0)
        pltpu.async_copy(st_v.at[b], acc_sh.at[idx_v.at[j]],
                         ssems[b], add=True)

    for j in (NITER - 2, NITER - 1):
        wait_scat(j % 2, j)

    plsc.subcore_barrier()

    pltpu.sync_copy(acc_sh.at[pl.ds(seg_base, SEG_S)], seg_v)
    pltpu.sync_copy(seg_v, out_hbm.at[c, pl.ds(seg_base, SEG_S)])


@functools.cache
def _make_sc_segment_sum():
    mesh = plsc.VectorSubcoreMesh(
        core_axis_name="c", subcore_axis_name="s",
        num_cores=NC, num_subcores=NS)
    return pl.kernel(
        _sc_body,
        out_type=jax.ShapeDtypeStruct((NC, NUM_GRAPHS, CH), jnp.float32),
        mesh=mesh,
        scratch_types=[
            pltpu.VMEM((NITER, CHUNK), jnp.int32),
            pltpu.VMEM((2, CHUNK, CH), jnp.float32),
            pltpu.VMEM((2, CHUNK, CH), jnp.float32),
            pltpu.VMEM((SEG_S, CH), jnp.float32),
            pltpu.VMEM_SHARED((NUM_GRAPHS, CH), jnp.float32),
            pltpu.SemaphoreType.DMA,
            pltpu.SemaphoreType.DMA,
            pltpu.SemaphoreType.DMA,
            pltpu.SemaphoreType.DMA,
        ],
    )


def _merge_body(p_ref, out_ref):
    out_ref[...] = p_ref[0] + p_ref[1]


def _tc_merge(partials):
    blk = 2048
    return pl.pallas_call(
        _merge_body,
        grid=(NUM_GRAPHS // blk,),
        in_specs=[pl.BlockSpec((NC, blk, CH), lambda i: (0, i, 0))],
        out_specs=pl.BlockSpec((blk, CH), lambda i: (i, 0)),
        out_shape=jax.ShapeDtypeStruct((NUM_GRAPHS, CH), jnp.float32),
    )(partials)


def kernel(input_rep, final_rep, graph_index, W_lin, b_lin, W_last, b_last):
    gi = graph_index.astype(jnp.int32)
    gi = jnp.pad(gi, (0, N_PAD - N_NODES)).reshape(SLABS, NW, NITER, CHUNK)
    bl = b_lin.reshape(1, CH)
    bb = b_last.reshape(1, CH)

    sc_segsum = _make_sc_segment_sum()
    partials = jnp.zeros((NC, NUM_GRAPHS, CH), jnp.float32)
    for k in range(SLABS):
        g_k = _tc_gated_slab(k, input_rep, final_rep, W_lin, bl, W_last, bb)
        partials = sc_segsum(g_k, gi[k], partials)
    return _tc_merge(partials)


# speedup vs baseline: 3.5114x; 1.0869x over previous
import functools

import jax
import jax.numpy as jnp
import numpy as np
from jax import lax
from jax.experimental import pallas as pl
from jax.experimental.pallas import tpu as pltpu
from jax.experimental.pallas import tpu_sc as plsc

N_NODES = 100000
NUM_GRAPHS = 4096
CH = 128

NC = 2
NS = 16
NW = NC * NS
N_PAD = 102400
SLABS = 4
SLAB_ROWS = N_PAD // SLABS
ROWS_W = SLAB_ROWS // NW
CHUNK = 80
NCHUNK = ROWS_W // CHUNK
NITER = NCHUNK // 2
SEG_S = NUM_GRAPHS // NS

BLK = 1600
PBLK = BLK // 2
BLOCKS_SLAB = SLAB_ROWS // BLK
LAST_BLK = N_NODES // BLK

_HI_MASK = np.uint32(0xFFFF0000)
_ROUND = np.uint32(0x8000)
_SH16 = np.uint32(16)
_IHI_MASK = np.int32(-65536)
_ISH16 = np.int32(16)


def _tc_body_for_slab(slab):
    def body(x1_ref, x2_ref, w1_ref, w2_ref, bl_ref, wl_ref, bb_ref, out_ref):
        gb = slab * BLOCKS_SLAB + pl.program_id(0)
        x1 = x1_ref[...].astype(jnp.bfloat16)
        x2 = x2_ref[...].astype(jnp.bfloat16)
        z = (jnp.dot(x1, w1_ref[...].astype(jnp.bfloat16),
                     preferred_element_type=jnp.float32)
             + jnp.dot(x2, w2_ref[...].astype(jnp.bfloat16),
                       preferred_element_type=jnp.float32)
             + bl_ref[...])
        h = (jnp.dot(x2, wl_ref[...].astype(jnp.bfloat16),
                     preferred_element_type=jnp.float32)
             + bb_ref[...])
        g = jax.nn.sigmoid(z) * h
        rows = gb * BLK + lax.broadcasted_iota(jnp.int32, (BLK, 1), 0)
        g = jnp.where(rows < N_NODES, g, 0.0)
        ulo = lax.bitcast_convert_type(g[:PBLK], jnp.uint32)
        uhi = lax.bitcast_convert_type(g[PBLK:], jnp.uint32)
        lo16 = lax.shift_right_logical(ulo + _ROUND, _SH16)
        hi16 = (uhi + _ROUND) & _HI_MASK
        out_ref[...] = lax.bitcast_convert_type(hi16 | lo16, jnp.float32)

    return body


def _tc_gated_slab(slab, input_rep, final_rep, W_lin, b_lin, w_last, b_last):
    clamp = lambda i: (jnp.minimum(slab * BLOCKS_SLAB + i, LAST_BLK), 0)
    full = lambda i: (0, 0)
    return pl.pallas_call(
        _tc_body_for_slab(slab),
        grid=(BLOCKS_SLAB,),
        in_specs=[
            pl.BlockSpec((BLK, CH), clamp),
            pl.BlockSpec((BLK, CH), clamp),
            pl.BlockSpec((CH, CH), full),
            pl.BlockSpec((CH, CH), lambda i: (1, 0)),
            pl.BlockSpec((1, CH), full),
            pl.BlockSpec((CH, CH), full),
            pl.BlockSpec((1, CH), full),
        ],
        out_specs=pl.BlockSpec((PBLK, CH), lambda i: (i, 0)),
        out_shape=jax.ShapeDtypeStruct((SLAB_ROWS // 2, CH), jnp.float32),
        compiler_params=pltpu.CompilerParams(
            dimension_semantics=("arbitrary",),
        ),
        name=f"tc_gated_slab{slab}",
    )(input_rep, final_rep, W_lin, W_lin, b_lin, w_last, b_last)


def _sc_body(g_hbm, gi_hbm, init_hbm, out_hbm,
             idx_v, pk_v, st_v, seg_v, acc_sh,
             semA, semB, semL0, semL1, semH0, semH1):
    c = lax.axis_index("c")
    s = lax.axis_index("s")
    w = c * NS + s
    seg_base = s * SEG_S

    pltpu.sync_copy(init_hbm.at[c, pl.ds(seg_base, SEG_S)], seg_v)
    pltpu.sync_copy(seg_v, acc_sh.at[pl.ds(seg_base, SEG_S)])

    pltpu.sync_copy(gi_hbm.at[w], idx_v)
    plsc.subcore_barrier()

    pbase = w * (NITER * CHUNK)
    mask = jnp.full((16,), _IHI_MASK, jnp.int32)
    gsems = (semA, semB)
    lsems = (semL0, semL1)
    hsems = (semH0, semH1)

    def start_g(j, b):
        pltpu.async_copy(
            g_hbm.at[pl.ds(pbase + j * CHUNK, CHUNK)], pk_v.at[b], gsems[b])

    def wait_g(j, b):
        pltpu.make_async_copy(
            g_hbm.at[pl.ds(pbase + j * CHUNK, CHUNK)], pk_v.at[b],
            gsems[b]).wait()

    def wait_scat(b, j, half, sem):
        pltpu.make_async_copy(
            st_v.at[b, half], acc_sh.at[idx_v.at[j]], sem).wait()

    start_g(0, 0)
    for j in range(NITER):
        b = j % 2
        wait_g(j, b)
        if j + 1 < NITER:
            start_g(j + 1, 1 - b)
        if j >= 2:
            wait_scat(b, j - 2, 0, lsems[b])
            wait_scat(b, (j - 2) + NITER, 1, hsems[b])

        def unpack_row(p, carry):
            for c4 in range(CH // 16):
                u = lax.bitcast_convert_type(
                    pk_v[b, p, pl.ds(c4 * 16, 16)], jnp.int32)
                lo = lax.bitcast_convert_type(
                    lax.shift_left(u, _ISH16), jnp.float32)
                hi = lax.bitcast_convert_type(u & mask, jnp.float32)
                st_v[b, 0, p, pl.ds(c4 * 16, 16)] = lo
                st_v[b, 1, p, pl.ds(c4 * 16, 16)] = hi
            return carry

        lax.fori_loop(0, CHUNK, unpack_row, 0)
        pltpu.async_copy(st_v.at[b, 0], acc_sh.at[idx_v.at[j]],
                         lsems[b], add=True)
        pltpu.async_copy(st_v.at[b, 1], acc_sh.at[idx_v.at[j + NITER]],
                         hsems[b], add=True)

    for j in (NITER - 2, NITER - 1):
        b = j % 2
        wait_scat(b, j, 0, lsems[b])
        wait_scat(b, j + NITER, 1, hsems[b])

    plsc.subcore_barrier()

    pltpu.sync_copy(acc_sh.at[pl.ds(seg_base, SEG_S)], seg_v)
    pltpu.sync_copy(seg_v, out_hbm.at[c, pl.ds(seg_base, SEG_S)])


@functools.cache
def _make_sc_segment_sum():
    mesh = plsc.VectorSubcoreMesh(
        core_axis_name="c", subcore_axis_name="s",
        num_cores=NC, num_subcores=NS)
    return pl.kernel(
        _sc_body,
        out_type=jax.ShapeDtypeStruct((NC, NUM_GRAPHS, CH), jnp.float32),
        mesh=mesh,
        scratch_types=[
            pltpu.VMEM((NCHUNK, CHUNK), jnp.int32),
            pltpu.VMEM((2, CHUNK, CH), jnp.float32),
            pltpu.VMEM((2, 2, CHUNK, CH), jnp.float32),
            pltpu.VMEM((SEG_S, CH), jnp.float32),
            pltpu.VMEM_SHARED((NUM_GRAPHS, CH), jnp.float32),
            pltpu.SemaphoreType.DMA,
            pltpu.SemaphoreType.DMA,
            pltpu.SemaphoreType.DMA,
            pltpu.SemaphoreType.DMA,
            pltpu.SemaphoreType.DMA,
            pltpu.SemaphoreType.DMA,
        ],
    )


def _merge_body(p_ref, out_ref):
    out_ref[...] = p_ref[0] + p_ref[1]


def _tc_merge(partials):
    blk = 2048
    return pl.pallas_call(
        _merge_body,
        grid=(NUM_GRAPHS // blk,),
        in_specs=[pl.BlockSpec((NC, blk, CH), lambda i: (0, i, 0))],
        out_specs=pl.BlockSpec((blk, CH), lambda i: (i, 0)),
        out_shape=jax.ShapeDtypeStruct((NUM_GRAPHS, CH), jnp.float32),
    )(partials)


_CHUNK_ORDER = None


def _chunk_order():
    global _CHUNK_ORDER
    if _CHUNK_ORDER is None:
        order = np.zeros((NW, NCHUNK), np.int32)
        for v in range(NW):
            b, h = v // 2, v % 2
            for j in range(NCHUNK):
                if j < NITER:
                    order[v, j] = b * 20 + h * NITER + j
                else:
                    order[v, j] = b * 20 + 10 + h * NITER + (j - NITER)
        _CHUNK_ORDER = order.reshape(-1)
    return _CHUNK_ORDER


def kernel(input_rep, final_rep, graph_index, W_lin, b_lin, W_last, b_last):
    gi = graph_index.astype(jnp.int32)
    gi = jnp.pad(gi, (0, N_PAD - N_NODES))
    gi = gi.reshape(SLABS, SLAB_ROWS // CHUNK, CHUNK)
    gi = gi[:, _chunk_order()].reshape(SLABS, NW, NCHUNK, CHUNK)
    bl = b_lin.reshape(1, CH)
    bb = b_last.reshape(1, CH)

    sc_segsum = _make_sc_segment_sum()
    partials = jnp.zeros((NC, NUM_GRAPHS, CH), jnp.float32)
    for k in range(SLABS):
        g_k = _tc_gated_slab(k, input_rep, final_rep, W_lin, bl, W_last, bb)
        partials = sc_segsum(g_k, gi[k], partials)
    return _tc_merge(partials)


# speedup vs baseline: 3.8516x; 1.0969x over previous
import functools

import jax
import jax.numpy as jnp
import numpy as np
from jax import lax
from jax.experimental import pallas as pl
from jax.experimental.pallas import tpu as pltpu
from jax.experimental.pallas import tpu_sc as plsc

N_NODES = 100000
NUM_GRAPHS = 4096
CH = 128

NC = 2
NS = 16
NW = NC * NS
N_PAD = 102400
SLABS = 4
SLAB_ROWS = N_PAD // SLABS
ROWS_W = SLAB_ROWS // NW
CHUNK = 80
NCHUNK = ROWS_W // CHUNK
NITER = NCHUNK // 2
SEG_S = NUM_GRAPHS // NS

BLK = 6400
PBLK = BLK // 2
WPB = 8
BLOCKS_SLAB = SLAB_ROWS // BLK
LAST_BLK = N_NODES // BLK

_HI_MASK = np.uint32(0xFFFF0000)
_ROUND = np.uint32(0x8000)
_SH16 = np.uint32(16)
_IHI_MASK = np.int32(-65536)
_ISH16 = np.int32(16)


def _tc_body_for_slab(slab):
    def body(x1_ref, x2_ref, w1_ref, w2_ref, bl_ref, wl_ref, bb_ref, out_ref):
        gb = slab * BLOCKS_SLAB + pl.program_id(0)
        x1 = x1_ref[...].astype(jnp.bfloat16)
        x2 = x2_ref[...].astype(jnp.bfloat16)
        z = (jnp.dot(x1, w1_ref[...].astype(jnp.bfloat16),
                     preferred_element_type=jnp.float32)
             + jnp.dot(x2, w2_ref[...].astype(jnp.bfloat16),
                       preferred_element_type=jnp.float32)
             + bl_ref[...])
        h = (jnp.dot(x2, wl_ref[...].astype(jnp.bfloat16),
                     preferred_element_type=jnp.float32)
             + bb_ref[...])
        g = jax.nn.sigmoid(z) * h
        rows = gb * BLK + lax.broadcasted_iota(jnp.int32, (BLK, 1), 0)
        g = jnp.where(rows < N_NODES, g, 0.0)
        ulo = lax.bitcast_convert_type(g[:PBLK], jnp.uint32)
        uhi = lax.bitcast_convert_type(g[PBLK:], jnp.uint32)
        lo16 = lax.shift_right_logical(ulo + _ROUND, _SH16)
        hi16 = (uhi + _ROUND) & _HI_MASK
        out_ref[...] = lax.bitcast_convert_type(hi16 | lo16, jnp.float32)

    return body


def _tc_gated_slab(slab, input_rep, final_rep, W_lin, b_lin, w_last, b_last):
    clamp = lambda i: (jnp.minimum(slab * BLOCKS_SLAB + i, LAST_BLK), 0)
    full = lambda i: (0, 0)
    return pl.pallas_call(
        _tc_body_for_slab(slab),
        grid=(BLOCKS_SLAB,),
        in_specs=[
            pl.BlockSpec((BLK, CH), clamp),
            pl.BlockSpec((BLK, CH), clamp),
            pl.BlockSpec((CH, CH), full),
            pl.BlockSpec((CH, CH), lambda i: (1, 0)),
            pl.BlockSpec((1, CH), full),
            pl.BlockSpec((CH, CH), full),
            pl.BlockSpec((1, CH), full),
        ],
        out_specs=pl.BlockSpec((PBLK, CH), lambda i: (i, 0)),
        out_shape=jax.ShapeDtypeStruct((SLAB_ROWS // 2, CH), jnp.float32),
        compiler_params=pltpu.CompilerParams(
            dimension_semantics=("arbitrary",),
        ),
        name=f"tc_gated_slab{slab}",
    )(input_rep, final_rep, W_lin, W_lin, b_lin, w_last, b_last)


def _sc_body(g_hbm, gi_hbm, init_hbm, out_hbm,
             idx_v, pk_v, st_v, seg_v, acc_sh,
             semA, semB, semL0, semL1, semH0, semH1, semI):
    c = lax.axis_index("c")
    s = lax.axis_index("s")
    w = c * NS + s
    seg_base = s * SEG_S

    pltpu.sync_copy(init_hbm.at[c, pl.ds(seg_base, SEG_S)], seg_v)
    pltpu.sync_copy(seg_v, acc_sh.at[pl.ds(seg_base, SEG_S)])

    q = w // WPB
    r = w - WPB * q
    lo_off = q * BLK + r * (NITER * CHUNK)
    hi_off = lo_off + PBLK
    for j in range(NITER):
        pltpu.async_copy(
            gi_hbm.at[pl.ds(lo_off + j * CHUNK, CHUNK)], idx_v.at[j], semI)
        pltpu.async_copy(
            gi_hbm.at[pl.ds(hi_off + j * CHUNK, CHUNK)],
            idx_v.at[j + NITER], semI)
    for j in range(NITER):
        pltpu.make_async_copy(
            gi_hbm.at[pl.ds(lo_off + j * CHUNK, CHUNK)], idx_v.at[j],
            semI).wait()
        pltpu.make_async_copy(
            gi_hbm.at[pl.ds(hi_off + j * CHUNK, CHUNK)],
            idx_v.at[j + NITER], semI).wait()
    plsc.subcore_barrier()

    pbase = w * (NITER * CHUNK)
    mask = jnp.full((16,), _IHI_MASK, jnp.int32)
    gsems = (semA, semB)
    lsems = (semL0, semL1)
    hsems = (semH0, semH1)

    def start_g(j, b):
        pltpu.async_copy(
            g_hbm.at[pl.ds(pbase + j * CHUNK, CHUNK)], pk_v.at[b], gsems[b])

    def wait_g(j, b):
        pltpu.make_async_copy(
            g_hbm.at[pl.ds(pbase + j * CHUNK, CHUNK)], pk_v.at[b],
            gsems[b]).wait()

    def wait_scat(b, j, half, sem):
        pltpu.make_async_copy(
            st_v.at[b, half], acc_sh.at[idx_v.at[j]], sem).wait()

    start_g(0, 0)
    for j in range(NITER):
        b = j % 2
        wait_g(j, b)
        if j + 1 < NITER:
            start_g(j + 1, 1 - b)
        if j >= 2:
            wait_scat(b, j - 2, 0, lsems[b])
            wait_scat(b, (j - 2) + NITER, 1, hsems[b])

        def unpack_row(p, carry):
            for c4 in range(CH // 16):
                u = lax.bitcast_convert_type(
                    pk_v[b, p, pl.ds(c4 * 16, 16)], jnp.int32)
                lo = lax.bitcast_convert_type(
                    lax.shift_left(u, _ISH16), jnp.float32)
                hi = lax.bitcast_convert_type(u & mask, jnp.float32)
                st_v[b, 0, p, pl.ds(c4 * 16, 16)] = lo
                st_v[b, 1, p, pl.ds(c4 * 16, 16)] = hi
            return carry

        lax.fori_loop(0, CHUNK, unpack_row, 0)
        pltpu.async_copy(st_v.at[b, 0], acc_sh.at[idx_v.at[j]],
                         lsems[b], add=True)
        pltpu.async_copy(st_v.at[b, 1], acc_sh.at[idx_v.at[j + NITER]],
                         hsems[b], add=True)

    for j in (NITER - 2, NITER - 1):
        b = j % 2
        wait_scat(b, j, 0, lsems[b])
        wait_scat(b, j + NITER, 1, hsems[b])

    plsc.subcore_barrier()

    pltpu.sync_copy(acc_sh.at[pl.ds(seg_base, SEG_S)], seg_v)
    pltpu.sync_copy(seg_v, out_hbm.at[c, pl.ds(seg_base, SEG_S)])


@functools.cache
def _make_sc_segment_sum():
    mesh = plsc.VectorSubcoreMesh(
        core_axis_name="c", subcore_axis_name="s",
        num_cores=NC, num_subcores=NS)
    return pl.kernel(
        _sc_body,
        out_type=jax.ShapeDtypeStruct((NC, NUM_GRAPHS, CH), jnp.float32),
        mesh=mesh,
        scratch_types=[
            pltpu.VMEM((NCHUNK, CHUNK), jnp.int32),
            pltpu.VMEM((2, CHUNK, CH), jnp.float32),
            pltpu.VMEM((2, 2, CHUNK, CH), jnp.float32),
            pltpu.VMEM((SEG_S, CH), jnp.float32),
            pltpu.VMEM_SHARED((NUM_GRAPHS, CH), jnp.float32),
            pltpu.SemaphoreType.DMA,
            pltpu.SemaphoreType.DMA,
            pltpu.SemaphoreType.DMA,
            pltpu.SemaphoreType.DMA,
            pltpu.SemaphoreType.DMA,
            pltpu.SemaphoreType.DMA,
            pltpu.SemaphoreType.DMA,
        ],
    )


def _merge_body(p_ref, out_ref):
    out_ref[...] = p_ref[0] + p_ref[1]


def _tc_merge(partials):
    blk = 2048
    return pl.pallas_call(
        _merge_body,
        grid=(NUM_GRAPHS // blk,),
        in_specs=[pl.BlockSpec((NC, blk, CH), lambda i: (0, i, 0))],
        out_specs=pl.BlockSpec((blk, CH), lambda i: (i, 0)),
        out_shape=jax.ShapeDtypeStruct((NUM_GRAPHS, CH), jnp.float32),
    )(partials)


def kernel(input_rep, final_rep, graph_index, W_lin, b_lin, W_last, b_last):
    gi = graph_index.astype(jnp.int32)
    gi = jnp.pad(gi, (0, N_PAD - N_NODES))
    bl = b_lin.reshape(1, CH)
    bb = b_last.reshape(1, CH)

    sc_segsum = _make_sc_segment_sum()
    partials = jnp.zeros((NC, NUM_GRAPHS, CH), jnp.float32)
    for k in range(SLABS):
        g_k = _tc_gated_slab(k, input_rep, final_rep, W_lin, bl, W_last, bb)
        gi_k = lax.slice(gi, (k * SLAB_ROWS,), ((k + 1) * SLAB_ROWS,))
        partials = sc_segsum(g_k, gi_k, partials)
    return _tc_merge(partials)


# speedup vs baseline: 3.8573x; 1.0015x over previous
import functools

import jax
import jax.numpy as jnp
import numpy as np
from jax import lax
from jax.experimental import pallas as pl
from jax.experimental.pallas import tpu as pltpu
from jax.experimental.pallas import tpu_sc as plsc

N_NODES = 100000
NUM_GRAPHS = 4096
CH = 128

NC = 2
NS = 16
NW = NC * NS
N_PAD = 102400
SLABS = 4
SLAB_ROWS = N_PAD // SLABS
ROWS_W = SLAB_ROWS // NW
CHUNK = 80
NCHUNK = ROWS_W // CHUNK
NITER = NCHUNK // 2
SEG_S = NUM_GRAPHS // NS

BLK = 6400
PBLK = BLK // 2
WPB = 8
BLOCKS_SLAB = SLAB_ROWS // BLK
LAST_BLK = N_NODES // BLK

_HI_MASK = np.uint32(0xFFFF0000)
_ROUND = np.uint32(0x8000)
_SH16 = np.uint32(16)
_IHI_MASK = np.int32(-65536)
_ISH16 = np.int32(16)


def _tc_body_for_slab(slab):
    def body(x1_ref, x2_ref, w1_ref, w2_ref, bl_ref, wl_ref, bb_ref, out_ref):
        gb = slab * BLOCKS_SLAB + pl.program_id(0)
        x1 = x1_ref[...].astype(jnp.bfloat16)
        x2 = x2_ref[...].astype(jnp.bfloat16)
        z = (jnp.dot(x1, w1_ref[...].astype(jnp.bfloat16),
                     preferred_element_type=jnp.float32)
             + jnp.dot(x2, w2_ref[...].astype(jnp.bfloat16),
                       preferred_element_type=jnp.float32)
             + bl_ref[...])
        h = (jnp.dot(x2, wl_ref[...].astype(jnp.bfloat16),
                     preferred_element_type=jnp.float32)
             + bb_ref[...])
        g = jax.nn.sigmoid(z) * h
        rows = gb * BLK + lax.broadcasted_iota(jnp.int32, (BLK, 1), 0)
        g = jnp.where(rows < N_NODES, g, 0.0)
        ulo = lax.bitcast_convert_type(g[:PBLK], jnp.uint32)
        uhi = lax.bitcast_convert_type(g[PBLK:], jnp.uint32)
        lo16 = lax.shift_right_logical(ulo + _ROUND, _SH16)
        hi16 = (uhi + _ROUND) & _HI_MASK
        out_ref[...] = lax.bitcast_convert_type(hi16 | lo16, jnp.float32)

    return body


def _tc_gated_slab(slab, input_rep, final_rep, W_lin, b_lin, w_last, b_last):
    clamp = lambda i: (jnp.minimum(slab * BLOCKS_SLAB + i, LAST_BLK), 0)
    full = lambda i: (0, 0)
    return pl.pallas_call(
        _tc_body_for_slab(slab),
        grid=(BLOCKS_SLAB,),
        in_specs=[
            pl.BlockSpec((BLK, CH), clamp),
            pl.BlockSpec((BLK, CH), clamp),
            pl.BlockSpec((CH, CH), full),
            pl.BlockSpec((CH, CH), lambda i: (1, 0)),
            pl.BlockSpec((1, CH), full),
            pl.BlockSpec((CH, CH), full),
            pl.BlockSpec((1, CH), full),
        ],
        out_specs=pl.BlockSpec((PBLK, CH), lambda i: (i, 0)),
        out_shape=jax.ShapeDtypeStruct((SLAB_ROWS // 2, CH), jnp.float32),
        compiler_params=pltpu.CompilerParams(
            dimension_semantics=("arbitrary",),
        ),
        name=f"tc_gated_slab{slab}",
    )(input_rep, final_rep, W_lin, W_lin, b_lin, w_last, b_last)


def _sc_body(g_hbm, gi_hbm, init_hbm, out_hbm,
             idx_v, pk_v, st_v, seg_v, acc_sh,
             semA, semB, semL0, semL1, semH0, semH1, semI):
    c = lax.axis_index("c")
    s = lax.axis_index("s")
    w = c * NS + s
    seg_base = s * SEG_S

    pltpu.sync_copy(init_hbm.at[c, pl.ds(seg_base, SEG_S)], seg_v)
    pltpu.sync_copy(seg_v, acc_sh.at[pl.ds(seg_base, SEG_S)])

    q = w // WPB
    r = w - WPB * q
    lo_off = q * BLK + r * (NITER * CHUNK)
    hi_off = lo_off + PBLK
    for j in range(NITER):
        pltpu.async_copy(
            gi_hbm.at[pl.ds(lo_off + j * CHUNK, CHUNK)], idx_v.at[j], semI)
        pltpu.async_copy(
            gi_hbm.at[pl.ds(hi_off + j * CHUNK, CHUNK)],
            idx_v.at[j + NITER], semI)
    for j in range(NITER):
        pltpu.make_async_copy(
            gi_hbm.at[pl.ds(lo_off + j * CHUNK, CHUNK)], idx_v.at[j],
            semI).wait()
        pltpu.make_async_copy(
            gi_hbm.at[pl.ds(hi_off + j * CHUNK, CHUNK)],
            idx_v.at[j + NITER], semI).wait()
    plsc.subcore_barrier()

    pbase = w * (NITER * CHUNK)
    mask = jnp.full((16,), _IHI_MASK, jnp.int32)
    gsems = (semA, semB)
    lsems = (semL0, semL1)
    hsems = (semH0, semH1)

    def start_g(j, b):
        pltpu.async_copy(
            g_hbm.at[pl.ds(pbase + j * CHUNK, CHUNK)], pk_v.at[b], gsems[b])

    def wait_g(j, b):
        pltpu.make_async_copy(
            g_hbm.at[pl.ds(pbase + j * CHUNK, CHUNK)], pk_v.at[b],
            gsems[b]).wait()

    def wait_scat(b, j, half, sem):
        pltpu.make_async_copy(
            st_v.at[b, half], acc_sh.at[idx_v.at[j]], sem).wait()

    start_g(0, 0)
    for j in range(NITER):
        b = j % 2
        wait_g(j, b)
        if j + 1 < NITER:
            start_g(j + 1, 1 - b)
        if j >= 2:
            wait_scat(b, j - 2, 0, lsems[b])
            wait_scat(b, (j - 2) + NITER, 1, hsems[b])

        def unpack_row(p, carry):
            for c4 in range(CH // 16):
                u = lax.bitcast_convert_type(
                    pk_v[b, p, pl.ds(c4 * 16, 16)], jnp.int32)
                lo = lax.bitcast_convert_type(
                    lax.shift_left(u, _ISH16), jnp.float32)
                hi = lax.bitcast_convert_type(u & mask, jnp.float32)
                st_v[b, 0, p, pl.ds(c4 * 16, 16)] = lo
                st_v[b, 1, p, pl.ds(c4 * 16, 16)] = hi
            return carry

        lax.fori_loop(0, CHUNK, unpack_row, 0)
        pltpu.async_copy(st_v.at[b, 0], acc_sh.at[idx_v.at[j]],
                         lsems[b], add=True)
        pltpu.async_copy(st_v.at[b, 1], acc_sh.at[idx_v.at[j + NITER]],
                         hsems[b], add=True)

    for j in (NITER - 2, NITER - 1):
        b = j % 2
        wait_scat(b, j, 0, lsems[b])
        wait_scat(b, j + NITER, 1, hsems[b])

    plsc.subcore_barrier()

    pltpu.sync_copy(acc_sh.at[pl.ds(seg_base, SEG_S)], seg_v)
    pltpu.sync_copy(seg_v, out_hbm.at[c, pl.ds(seg_base, SEG_S)])


@functools.cache
def _make_sc_segment_sum():
    mesh = plsc.VectorSubcoreMesh(
        core_axis_name="c", subcore_axis_name="s",
        num_cores=NC, num_subcores=NS)
    return pl.kernel(
        _sc_body,
        out_type=jax.ShapeDtypeStruct((NC, NUM_GRAPHS, CH), jnp.float32),
        mesh=mesh,
        scratch_types=[
            pltpu.VMEM((NCHUNK, CHUNK), jnp.int32),
            pltpu.VMEM((2, CHUNK, CH), jnp.float32),
            pltpu.VMEM((2, 2, CHUNK, CH), jnp.float32),
            pltpu.VMEM((SEG_S, CH), jnp.float32),
            pltpu.VMEM_SHARED((NUM_GRAPHS, CH), jnp.float32),
            pltpu.SemaphoreType.DMA,
            pltpu.SemaphoreType.DMA,
            pltpu.SemaphoreType.DMA,
            pltpu.SemaphoreType.DMA,
            pltpu.SemaphoreType.DMA,
            pltpu.SemaphoreType.DMA,
            pltpu.SemaphoreType.DMA,
        ],
    )


def _merge_body(p_ref, out_ref):
    out_ref[...] = p_ref[0] + p_ref[1]


def _tc_merge(partials):
    blk = 2048
    return pl.pallas_call(
        _merge_body,
        grid=(NUM_GRAPHS // blk,),
        in_specs=[pl.BlockSpec((NC, blk, CH), lambda i: (0, i, 0))],
        out_specs=pl.BlockSpec((blk, CH), lambda i: (i, 0)),
        out_shape=jax.ShapeDtypeStruct((NUM_GRAPHS, CH), jnp.float32),
    )(partials)


def kernel(input_rep, final_rep, graph_index, W_lin, b_lin, W_last, b_last):
    gi = graph_index.astype(jnp.int32)
    gi = jnp.pad(gi, (0, N_PAD - N_NODES))
    bl = b_lin.reshape(1, CH)
    bb = b_last.reshape(1, CH)

    sc_segsum = _make_sc_segment_sum()
    partials = jnp.asarray(np.zeros((NC, NUM_GRAPHS, CH), np.float32))
    for k in range(SLABS):
        g_k = _tc_gated_slab(k, input_rep, final_rep, W_lin, bl, W_last, bb)
        gi_k = lax.slice(gi, (k * SLAB_ROWS,), ((k + 1) * SLAB_ROWS,))
        partials = sc_segsum(g_k, gi_k, partials)
    return _tc_merge(partials)
